# baseline (speedup 1.0000x reference)
import functools

import jax
import jax.numpy as jnp
from jax import lax
from jax.experimental import pallas as pl
from jax.experimental.pallas import tpu as pltpu
from jax.experimental.pallas import tpu_sc as plsc

BATCH = 16384
EMB = 64
ROWW = 128
_info = plsc.get_sparse_core_info()
NC, NS, L = _info.num_cores, _info.num_subcores, _info.num_lanes
NW = NC * NS
BPW = BATCH // NW
CHUNK = 128
NCHUNK = BPW // CHUNK


def _make_sc_kernel():
  mesh = plsc.VectorSubcoreMesh(core_axis_name="c", subcore_axis_name="s")

  @functools.partial(
      pl.kernel,
      mesh=mesh,
      compiler_params=pltpu.CompilerParams(
          needs_layout_passes=False, use_tc_tiling_on_sc=True),
      out_type=jax.ShapeDtypeStruct((BATCH,), jnp.float32),
      scratch_types=[
          pltpu.VMEM((NCHUNK, CHUNK), jnp.int32),
          pltpu.VMEM((NCHUNK, CHUNK), jnp.int32),
          pltpu.VMEM((BPW,), jnp.int32),
          pltpu.VMEM((BPW,), jnp.int32),
          pltpu.VMEM((CHUNK, ROWW), jnp.float32),
          pltpu.VMEM((CHUNK, ROWW), jnp.float32),
          pltpu.VMEM((BPW,), jnp.float32),
          pltpu.SemaphoreType.DMA,
          pltpu.SemaphoreType.DMA,
      ],
  )
  def k(ur_hbm, vr_hbm, uo_hbm, vo_hbm, ue_hbm, ve_hbm, out_hbm,
        u_idx, v_idx, u_off, v_off, u_rows, v_rows, out_v, sem_u, sem_v):
    wid = lax.axis_index("s") * NC + lax.axis_index("c")
    base = wid * BPW

    pltpu.sync_copy(ur_hbm.at[pl.ds(wid * NCHUNK, NCHUNK)], u_idx)
    pltpu.sync_copy(vr_hbm.at[pl.ds(wid * NCHUNK, NCHUNK)], v_idx)
    pltpu.sync_copy(uo_hbm.at[pl.ds(base, BPW)], u_off)
    pltpu.sync_copy(vo_hbm.at[pl.ds(base, BPW)], v_off)

    def chunk_body(j, _):
      cu = pltpu.async_copy(ue_hbm.at[u_idx.at[j]], u_rows, sem_u)
      cv = pltpu.async_copy(ve_hbm.at[v_idx.at[j]], v_rows, sem_v)
      cu.wait()
      cv.wait()

      def body(g, __):
        rows = g * L + lax.iota(jnp.int32, L)
        uo = u_off[pl.ds(j * CHUNK + g * L, L)]
        vo = v_off[pl.ds(j * CHUNK + g * L, L)]
        acc = jnp.zeros((L,), jnp.float32)
        for d in range(EMB):
          a = plsc.load_gather(u_rows, [rows, uo + d])
          b = plsc.load_gather(v_rows, [rows, vo + d])
          acc = acc + a * b
        out_v[pl.ds(j * CHUNK + g * L, L)] = acc
        return __

      lax.fori_loop(0, CHUNK // L, body, None)
      return _

    lax.fori_loop(0, NCHUNK, chunk_body, None)

    pltpu.sync_copy(out_v, out_hbm.at[pl.ds(base, BPW)])

  return k


_sc_kernel = _make_sc_kernel()


def kernel(u, v, user_emb, item_emb):
  u = u.astype(jnp.int32)
  v = v.astype(jnp.int32)
  ue2 = user_emb.reshape(500000, ROWW)
  ve2 = item_emb.reshape(500000, ROWW)
  ur = (u >> 1).reshape(NW * NCHUNK, CHUNK)
  vr = (v >> 1).reshape(NW * NCHUNK, CHUNK)
  uo = (u & 1) * EMB
  vo = (v & 1) * EMB
  return _sc_kernel(ur, vr, uo, vo, ue2, ve2)
